# baseline (speedup 1.0000x reference)
import functools

import jax
import jax.numpy as jnp
from jax import lax
from jax.experimental import pallas as pl
from jax.experimental.pallas import tpu as pltpu
from jax.experimental.pallas import tpu_sc as plsc

NB = 64
NC = 2
NS = 16
NW = NC * NS
TASKS = NB * 16
TPW = TASKS // NW


def _qpool_body(rho_hbm, out_hbm, in_ab, in_cd, outb, sem_in):
    wid = lax.axis_index("s") * NC + lax.axis_index("c")
    iota = lax.iota(jnp.int32, 16)
    zeros = jnp.zeros((16,), jnp.int32)
    ones = jnp.ones((16,), jnp.int32)

    def task(i, carry):
        t = wid * TPW + i
        b = t // 16
        p = t % 16
        h_ab = pltpu.async_copy(
            rho_hbm.at[b, p, 0, :, :, :, pl.ds(0, 16), :], in_ab, sem_in)
        h_cd = pltpu.async_copy(
            rho_hbm.at[b, p, 1, :, :, p, pl.ds(16, 16), :], in_cd, sem_in)
        h_ab.wait()
        h_cd.wait()

        for q in range(16):
            qs = jnp.full((16,), q, jnp.int32)
            for r in range(16):
                avec = plsc.load_gather(
                    in_ab, [qs, zeros, jnp.full((16,), r, jnp.int32), iota, zeros])
                outb[q, pl.ds(r * 16, 16)] = avec
            bvec = plsc.load_gather(in_ab, [qs, ones, iota, qs, ones])
            plsc.addupdate_scatter(outb, [qs, iota * 16 + q], bvec)
            cvec = plsc.load_gather(in_cd, [qs, zeros, iota, zeros])
            dvec = plsc.load_gather(in_cd, [qs, ones, qs, ones])
            cvec = cvec + jnp.where(iota == q, dvec, jnp.zeros((16,), jnp.float32))
            cur = outb[q, pl.ds(p * 16, 16)]
            outb[q, pl.ds(p * 16, 16)] = cur + cvec

        pltpu.sync_copy(outb, out_hbm.at[b, pl.ds(p * 16, 16), :])
        return carry

    lax.fori_loop(0, TPW, task, 0)


@functools.partial(
    pl.kernel,
    out_type=jax.ShapeDtypeStruct((NB, 256, 256), jnp.float32),
    mesh=plsc.VectorSubcoreMesh(core_axis_name="c", subcore_axis_name="s"),
    scratch_types=[
        pltpu.VMEM((16, 2, 16, 16, 2), jnp.float32),
        pltpu.VMEM((16, 2, 16, 2), jnp.float32),
        pltpu.VMEM((16, 256), jnp.float32),
        pltpu.SemaphoreType.DMA,
    ],
    compiler_params=pltpu.CompilerParams(
        use_tc_tiling_on_sc=False, needs_layout_passes=False),
)
def _qpool_sc(rho_hbm, out_hbm, in_ab, in_cd, outb, sem_in):
    _qpool_body(rho_hbm, out_hbm, in_ab, in_cd, outb, sem_in)


def kernel(rho):
    rho9 = rho.reshape(NB, 16, 2, 16, 2, 16, 32, 2)
    return _qpool_sc(rho9)


# speedup vs baseline: 1.0011x; 1.0011x over previous
import functools

import jax
import jax.numpy as jnp
from jax import lax
from jax.experimental import pallas as pl
from jax.experimental.pallas import tpu as pltpu
from jax.experimental.pallas import tpu_sc as plsc

NB = 64
NC = 2
NS = 16
NW = NC * NS
TASKS = NB * 16
TPW = TASKS // NW


def _qpool_body(rho_hbm, rho4_hbm, out_hbm, in_ab, in_cd, outb, sem_in):
    wid = lax.axis_index("s") * NC + lax.axis_index("c")
    iota = lax.iota(jnp.int32, 16)
    zeros = jnp.zeros((16,), jnp.int32)
    ones = jnp.ones((16,), jnp.int32)
    iota2 = iota * 2

    def task(i, carry):
        t = wid * TPW + i
        b = t // 16
        p = t % 16
        h_ab = pltpu.async_copy(rho_hbm.at[b, 2 * p, :, :], in_ab, sem_in)
        h_cd = pltpu.async_copy(
            rho4_hbm.at[b, p, 1, :, :, p, pl.ds(16, 16), :], in_cd, sem_in)
        h_ab.wait()
        h_cd.wait()

        for q in range(16):
            qs = jnp.full((16,), q, jnp.int32)
            row_a = jnp.full((16,), 2 * q, jnp.int32)
            for r in range(16):
                avec = plsc.load_gather(in_ab, [row_a, iota2 + 64 * r])
                outb[q, pl.ds(r * 16, 16)] = avec
            bvec = plsc.load_gather(
                in_ab, [row_a + 1, iota * 64 + (2 * q + 1)])
            plsc.addupdate_scatter(outb, [qs, iota * 16 + q], bvec)
            cvec = plsc.load_gather(in_cd, [qs, zeros, iota, zeros])
            dvec = plsc.load_gather(in_cd, [qs, ones, qs, ones])
            cvec = cvec + jnp.where(iota == q, dvec, jnp.zeros((16,), jnp.float32))
            cur = outb[q, pl.ds(p * 16, 16)]
            outb[q, pl.ds(p * 16, 16)] = cur + cvec

        pltpu.sync_copy(outb, out_hbm.at[b, pl.ds(p * 16, 16), :])
        return carry

    lax.fori_loop(0, TPW, task, 0)


@functools.partial(
    pl.kernel,
    out_type=jax.ShapeDtypeStruct((NB, 256, 256), jnp.float32),
    mesh=plsc.VectorSubcoreMesh(core_axis_name="c", subcore_axis_name="s"),
    scratch_types=[
        pltpu.VMEM((32, 1024), jnp.float32),
        pltpu.VMEM((16, 2, 16, 2), jnp.float32),
        pltpu.VMEM((16, 256), jnp.float32),
        pltpu.SemaphoreType.DMA,
    ],
    compiler_params=pltpu.CompilerParams(
        use_tc_tiling_on_sc=False, needs_layout_passes=False),
)
def _qpool_sc(rho_hbm, rho4_hbm, out_hbm, in_ab, in_cd, outb, sem_in):
    _qpool_body(rho_hbm, rho4_hbm, out_hbm, in_ab, in_cd, outb, sem_in)


def kernel(rho):
    rho3 = rho.reshape(NB, 32, 32, 1024)
    rho9 = rho.reshape(NB, 16, 2, 16, 2, 16, 32, 2)
    return _qpool_sc(rho3, rho9)


# speedup vs baseline: 254.8516x; 254.5790x over previous
import functools

import jax
import jax.numpy as jnp
from jax import lax
from jax.experimental import pallas as pl
from jax.experimental.pallas import tpu as pltpu
from jax.experimental.pallas import tpu_sc as plsc

NB = 64
NC = 2
NS = 16
NW = NC * NS
TASKS = NB * 16
TPW = TASKS // NW


def _qpool_body(rho_hbm, out_hbm, in_ab, in_cd, outb, sem_in):
    wid = lax.axis_index("s") * NC + lax.axis_index("c")
    iota = lax.iota(jnp.int32, 16)
    iota2 = iota * 2

    def task(i, carry):
        t = wid * TPW + i
        b = t // 16
        p = t % 16
        h_ab = pltpu.async_copy(rho_hbm.at[b, 2 * p, :, :], in_ab, sem_in)
        h_cd = pltpu.async_copy(
            rho_hbm.at[b, 2 * p + 1, :, pl.ds(64 * p + 32, 32)], in_cd, sem_in)
        h_ab.wait()
        h_cd.wait()

        for q in range(16):
            qs = jnp.full((16,), q, jnp.int32)
            row_a = jnp.full((16,), 2 * q, jnp.int32)
            for r in range(16):
                avec = plsc.load_gather(in_ab, [row_a, iota2 + 64 * r])
                outb[q, pl.ds(r * 16, 16)] = avec
            bvec = plsc.load_gather(
                in_ab, [row_a + 1, iota * 64 + (2 * q + 1)])
            plsc.addupdate_scatter(outb, [qs, iota * 16 + q], bvec)
            cvec = plsc.load_gather(in_cd, [row_a, iota2])
            dvec = plsc.load_gather(
                in_cd, [row_a + 1, jnp.full((16,), 2 * q + 1, jnp.int32)])
            cvec = cvec + jnp.where(iota == q, dvec, jnp.zeros((16,), jnp.float32))
            cur = outb[q, pl.ds(p * 16, 16)]
            outb[q, pl.ds(p * 16, 16)] = cur + cvec

        pltpu.sync_copy(outb, out_hbm.at[b, pl.ds(p * 16, 16), :])
        return carry

    lax.fori_loop(0, TPW, task, 0)


@functools.partial(
    pl.kernel,
    out_type=jax.ShapeDtypeStruct((NB, 256, 256), jnp.float32),
    mesh=plsc.VectorSubcoreMesh(core_axis_name="c", subcore_axis_name="s"),
    scratch_types=[
        pltpu.VMEM((32, 1024), jnp.float32),
        pltpu.VMEM((32, 32), jnp.float32),
        pltpu.VMEM((16, 256), jnp.float32),
        pltpu.SemaphoreType.DMA,
    ],
    compiler_params=pltpu.CompilerParams(
        use_tc_tiling_on_sc=False, needs_layout_passes=False),
)
def _qpool_sc(rho_hbm, out_hbm, in_ab, in_cd, outb, sem_in):
    _qpool_body(rho_hbm, out_hbm, in_ab, in_cd, outb, sem_in)


def kernel(rho):
    rho3 = rho.reshape(NB, 32, 32, 1024)
    return _qpool_sc(rho3)


# speedup vs baseline: 606.7424x; 2.3808x over previous
import functools

import jax
import jax.numpy as jnp
from jax import lax
from jax.experimental import pallas as pl
from jax.experimental.pallas import tpu as pltpu
from jax.experimental.pallas import tpu_sc as plsc

NB = 64
NC = 2
NS = 16
NW = NC * NS
TASKS = NB * 16
TPW = TASKS // NW


def _qpool_body(rho_hbm, out_hbm, in_ab, in_cd, outb, sem_in):
    wid = lax.axis_index("s") * NC + lax.axis_index("c")
    iota = lax.iota(jnp.int32, 16)
    iota2 = iota * 2

    def task(i, carry):
        t = wid * TPW + i
        b = t // 16
        p = t % 16
        h_ab = pltpu.async_copy(rho_hbm.at[b, 2 * p, :, :], in_ab, sem_in)
        cwin = (64 * p + 32) // 128 * 128
        co = (64 * p + 32) - cwin
        h_cd = pltpu.async_copy(
            rho_hbm.at[b, 2 * p + 1, :, pl.ds(cwin, 128)], in_cd, sem_in)
        h_ab.wait()
        h_cd.wait()

        for q in range(16):
            qs = jnp.full((16,), q, jnp.int32)
            row_a = jnp.full((16,), 2 * q, jnp.int32)
            for r in range(16):
                avec = plsc.load_gather(in_ab, [row_a, iota2 + 64 * r])
                outb[q, pl.ds(r * 16, 16)] = avec
            bvec = plsc.load_gather(
                in_ab, [row_a + 1, iota * 64 + (2 * q + 1)])
            plsc.addupdate_scatter(outb, [qs, iota * 16 + q], bvec)
            cvec = plsc.load_gather(in_cd, [row_a, iota2 + co])
            dvec = plsc.load_gather(
                in_cd, [row_a + 1, jnp.full((16,), 2 * q + 1, jnp.int32) + co])
            cvec = cvec + jnp.where(iota == q, dvec, jnp.zeros((16,), jnp.float32))
            cur = outb[q, pl.ds(p * 16, 16)]
            outb[q, pl.ds(p * 16, 16)] = cur + cvec

        pltpu.sync_copy(outb, out_hbm.at[b, pl.ds(p * 16, 16), :])
        return carry

    lax.fori_loop(0, TPW, task, 0)


@functools.partial(
    pl.kernel,
    out_type=jax.ShapeDtypeStruct((NB, 256, 256), jnp.float32),
    mesh=plsc.VectorSubcoreMesh(core_axis_name="c", subcore_axis_name="s"),
    scratch_types=[
        pltpu.VMEM((32, 1024), jnp.float32),
        pltpu.VMEM((32, 128), jnp.float32),
        pltpu.VMEM((16, 256), jnp.float32),
        pltpu.SemaphoreType.DMA,
    ],
    compiler_params=pltpu.CompilerParams(
        use_tc_tiling_on_sc=True, needs_layout_passes=False),
)
def _qpool_sc(rho_hbm, out_hbm, in_ab, in_cd, outb, sem_in):
    _qpool_body(rho_hbm, out_hbm, in_ab, in_cd, outb, sem_in)


def kernel(rho):
    rho3 = rho.reshape(NB, 32, 32, 1024)
    return _qpool_sc(rho3)


# speedup vs baseline: 873.8312x; 1.4402x over previous
import functools

import jax
import jax.numpy as jnp
from jax import lax
from jax.experimental import pallas as pl
from jax.experimental.pallas import tpu as pltpu
from jax.experimental.pallas import tpu_sc as plsc

NB = 64
NC = 2
NS = 16
NW = NC * NS
TASKS = NB * 16
TPW = TASKS // NW


def _qpool_body(rho_hbm, out_hbm,
                ab0, ab1, cd0, cd1, ob0, ob1, si0, si1, so0, so1):
    wid = lax.axis_index("s") * NC + lax.axis_index("c")
    iota = lax.iota(jnp.int32, 16)
    iota2 = iota * 2
    AB, CD, OB, SI, SO = (ab0, ab1), (cd0, cd1), (ob0, ob1), (si0, si1), (so0, so1)

    def bp(i):
        t = wid * TPW + i
        return t // 16, t % 16

    def in_descrs(i, s):
        b, p = bp(i)
        cwin = (64 * p + 32) // 128 * 128
        d1 = pltpu.make_async_copy(rho_hbm.at[b, 2 * p, :, :], AB[s], SI[s])
        d2 = pltpu.make_async_copy(
            rho_hbm.at[b, 2 * p + 1, :, pl.ds(cwin, 128)], CD[s], SI[s])
        return d1, d2

    def out_descr(i, s):
        b, p = bp(i)
        return pltpu.make_async_copy(
            OB[s], out_hbm.at[b, pl.ds(p * 16, 16), :], SO[s])

    def compute(i, s):
        in_ab, in_cd, outb = AB[s], CD[s], OB[s]
        _, p = bp(i)
        co = 32 + 64 * (p % 2)
        for q in range(16):
            qs = jnp.full((16,), q, jnp.int32)
            row_a = jnp.full((16,), 2 * q, jnp.int32)
            for r in range(16):
                avec = plsc.load_gather(in_ab, [row_a, iota2 + 64 * r])
                outb[q, pl.ds(r * 16, 16)] = avec
            bvec = plsc.load_gather(
                in_ab, [row_a + 1, iota * 64 + (2 * q + 1)])
            plsc.addupdate_scatter(outb, [qs, iota * 16 + q], bvec)
            cvec = plsc.load_gather(in_cd, [row_a, iota2 + co])
            dvec = plsc.load_gather(
                in_cd, [row_a + 1, jnp.full((16,), 2 * q + 1, jnp.int32) + co])
            cvec = cvec + jnp.where(iota == q, dvec, jnp.zeros((16,), jnp.float32))
            cur = outb[q, pl.ds(p * 16, 16)]
            outb[q, pl.ds(p * 16, 16)] = cur + cvec

    for s in range(2):
        d1, d2 = in_descrs(s, s)
        d1.start()
        d2.start()

    def pair(k, carry):
        for s in range(2):
            i = 2 * k + s
            d1, d2 = in_descrs(i, s)
            d1.wait()
            d2.wait()

            @pl.when(k > 0)
            def _wait_out():
                out_descr(i - 2, s).wait()

            compute(i, s)

            @pl.when(i + 2 < TPW)
            def _prefetch():
                e1, e2 = in_descrs(i + 2, s)
                e1.start()
                e2.start()

            out_descr(i, s).start()
        return carry

    lax.fori_loop(0, TPW // 2, pair, 0)

    for s in range(2):
        out_descr(TPW - 2 + s, s).wait()


@functools.partial(
    pl.kernel,
    out_type=jax.ShapeDtypeStruct((NB, 256, 256), jnp.float32),
    mesh=plsc.VectorSubcoreMesh(core_axis_name="c", subcore_axis_name="s"),
    scratch_types=[
        pltpu.VMEM((32, 1024), jnp.float32),
        pltpu.VMEM((32, 1024), jnp.float32),
        pltpu.VMEM((32, 128), jnp.float32),
        pltpu.VMEM((32, 128), jnp.float32),
        pltpu.VMEM((16, 256), jnp.float32),
        pltpu.VMEM((16, 256), jnp.float32),
        pltpu.SemaphoreType.DMA,
        pltpu.SemaphoreType.DMA,
        pltpu.SemaphoreType.DMA,
        pltpu.SemaphoreType.DMA,
    ],
    compiler_params=pltpu.CompilerParams(
        use_tc_tiling_on_sc=True, needs_layout_passes=False),
)
def _qpool_sc(rho_hbm, out_hbm,
              ab0, ab1, cd0, cd1, ob0, ob1, si0, si1, so0, so1):
    _qpool_body(rho_hbm, out_hbm,
                ab0, ab1, cd0, cd1, ob0, ob1, si0, si1, so0, so1)


def kernel(rho):
    rho3 = rho.reshape(NB, 32, 32, 1024)
    return _qpool_sc(rho3)
